# Initial kernel scaffold; baseline (speedup 1.0000x reference)
import jax
import jax.numpy as jnp
from jax.experimental import pallas as pl


def kernel(x, edge_index, We1, be1, We2, be2, Wd1, bd1, Wd2, bd2, Ws1, bs1, Ws2, bs2):
    raise NotImplementedError("write your pallas kernel here")


# speedup vs baseline: 4.6242x; 4.6242x over previous
import functools

import jax
import jax.numpy as jnp
from jax import lax
from jax.experimental import pallas as pl
from jax.experimental.pallas import tpu as pltpu
from jax.experimental.pallas import tpu_sc as plsc

N = 10000
D = 128
NC = 2
NS = 16
CHUNK = 128
CH = 40
EP = NC * NS * CH * CHUNK
ACC_ROWS = 10240
ZROWS = ACC_ROWS // NS
DROWS = ZROWS


@functools.lru_cache(maxsize=None)
def _sc_kernels():
    mesh = plsc.VectorSubcoreMesh(core_axis_name="c", subcore_axis_name="s",
                                  num_cores=NC, num_subcores=NS)

    @functools.partial(
        pl.kernel,
        out_type=jax.ShapeDtypeStruct((NC, ACC_ROWS, 16), jnp.float32),
        mesh=mesh,
        scratch_types=[
            pltpu.VMEM((CH, CHUNK), jnp.int32),
            pltpu.VMEM((CHUNK, 16), jnp.float32),
            pltpu.VMEM_SHARED((ACC_ROWS, 16), jnp.float32),
        ],
    )
    def sc_degree(dst_hbm, ones_hbm, zeros_hbm, out_hbm, dst_v, ones_v, acc):
        c = lax.axis_index("c")
        s = lax.axis_index("s")
        pltpu.sync_copy(zeros_hbm, acc.at[pl.ds(s * ZROWS, ZROWS)])
        pltpu.sync_copy(ones_hbm, ones_v)
        pltpu.sync_copy(dst_hbm.at[c, s], dst_v)
        plsc.subcore_barrier()

        def body(j, carry):
            pltpu.sync_copy(ones_v, acc.at[dst_v.at[j]], add=True)
            return carry

        lax.fori_loop(0, CH, body, 0)
        plsc.subcore_barrier()
        pltpu.sync_copy(acc.at[pl.ds(s * DROWS, DROWS)],
                        out_hbm.at[c, pl.ds(s * DROWS, DROWS)])

    @functools.partial(
        pl.kernel,
        out_type=jax.ShapeDtypeStruct((NC, ACC_ROWS, D), jnp.float32),
        mesh=mesh,
        scratch_types=[
            pltpu.VMEM((CH, CHUNK), jnp.int32),
            pltpu.VMEM((CH, CHUNK), jnp.int32),
            pltpu.VMEM((CHUNK, D), jnp.float32),
            pltpu.VMEM_SHARED((ACC_ROWS, D), jnp.float32),
            pltpu.SemaphoreType.DMA,
        ],
    )
    def sc_aggregate(g_hbm, src_hbm, dst_hbm, zeros_hbm, out_hbm,
                     src_v, dst_v, rows_v, acc, sem):
        c = lax.axis_index("c")
        s = lax.axis_index("s")
        pltpu.sync_copy(zeros_hbm, acc.at[pl.ds(s * ZROWS, ZROWS)])
        pltpu.sync_copy(src_hbm.at[c, s], src_v)
        pltpu.sync_copy(dst_hbm.at[c, s], dst_v)
        plsc.subcore_barrier()

        def body(j, carry):
            pltpu.async_copy(g_hbm.at[src_v.at[j]], rows_v, sem).wait()
            pltpu.sync_copy(rows_v, acc.at[dst_v.at[j]], add=True)
            return carry

        lax.fori_loop(0, CH, body, 0)
        plsc.subcore_barrier()
        pltpu.sync_copy(acc.at[pl.ds(s * DROWS, DROWS)],
                        out_hbm.at[c, pl.ds(s * DROWS, DROWS)])

    return sc_degree, sc_aggregate


BR = 1000
NRB = N // BR


def _dinv_block(degp_ref):
    deg = degp_ref[0, :, 0:1] + degp_ref[1, :, 0:1] + 1.0
    return lax.rsqrt(deg)


def _pre_body(h_ref, w_ref, degp_ref, g_ref):
    mm = jnp.dot(h_ref[...], w_ref[...], preferred_element_type=jnp.float32)
    g_ref[...] = _dinv_block(degp_ref) * mm


_pre = pl.pallas_call(
    _pre_body,
    grid=(NRB,),
    in_specs=[
        pl.BlockSpec((BR, D), lambda i: (i, 0)),
        pl.BlockSpec((D, D), lambda i: (0, 0)),
        pl.BlockSpec((NC, BR, 16), lambda i: (0, i, 0)),
    ],
    out_specs=pl.BlockSpec((BR, D), lambda i: (i, 0)),
    out_shape=jax.ShapeDtypeStruct((N, D), jnp.float32),
)


def _post_body(accp_ref, g_ref, b_ref, degp_ref, o_ref, *, relu):
    dinv = _dinv_block(degp_ref)
    v = dinv * (accp_ref[0] + accp_ref[1] + g_ref[...]) + b_ref[...]
    o_ref[...] = jnp.maximum(v, 0.0) if relu else v


def _make_post(relu):
    return pl.pallas_call(
        functools.partial(_post_body, relu=relu),
        grid=(NRB,),
        in_specs=[
            pl.BlockSpec((NC, BR, D), lambda i: (0, i, 0)),
            pl.BlockSpec((BR, D), lambda i: (i, 0)),
            pl.BlockSpec((1, D), lambda i: (0, 0)),
            pl.BlockSpec((NC, BR, 16), lambda i: (0, i, 0)),
        ],
        out_specs=pl.BlockSpec((BR, D), lambda i: (i, 0)),
        out_shape=jax.ShapeDtypeStruct((N, D), jnp.float32),
    )


_post_relu = _make_post(True)
_post_lin = _make_post(False)


def _acca_body(h_ref, wd1_ref, a_ref):
    @pl.when(pl.program_id(0) == 0)
    def _init():
        a_ref[...] = jnp.zeros_like(a_ref)

    a_ref[...] += lax.dot_general(
        h_ref[...], wd1_ref[...], (((0,), (0,)), ((), ())),
        preferred_element_type=jnp.float32)


_acca = pl.pallas_call(
    _acca_body,
    grid=(NRB,),
    in_specs=[
        pl.BlockSpec((BR, D), lambda i: (i, 0)),
        pl.BlockSpec((BR, D), lambda i: (i, 0)),
    ],
    out_specs=pl.BlockSpec((D, D), lambda i: (0, 0)),
    out_shape=jax.ShapeDtypeStruct((D, D), jnp.float32),
)


def _t_body(a_ref, bd1_ref, wd2_ref, bd2_ref, t_ref):
    u = jnp.maximum(a_ref[...] + bd1_ref[...], 0.0)
    t_ref[...] = jnp.dot(u, wd2_ref[...],
                         preferred_element_type=jnp.float32) + bd2_ref[...]


_tker = pl.pallas_call(
    _t_body,
    out_shape=jax.ShapeDtypeStruct((D, D), jnp.float32),
)


def _x_body(h_ref, t_ref, x_ref):
    x_ref[...] = jnp.dot(h_ref[...], t_ref[...],
                         preferred_element_type=jnp.float32)


_xker = pl.pallas_call(
    _x_body,
    grid=(NRB,),
    in_specs=[
        pl.BlockSpec((BR, D), lambda i: (i, 0)),
        pl.BlockSpec((D, D), lambda i: (0, 0)),
    ],
    out_specs=pl.BlockSpec((BR, D), lambda i: (i, 0)),
    out_shape=jax.ShapeDtypeStruct((N, D), jnp.float32),
)


SBR = 1000
SBC = 1024
NSR = N // SBR
NSC = -(-N // SBC)


def _s_body(hsr_ref, hsc_ref, o_ref):
    logits = lax.dot_general(
        hsr_ref[...], hsc_ref[...], (((1,), (1,)), ((), ())),
        preferred_element_type=jnp.float32)
    o_ref[...] = jax.nn.sigmoid(logits)


_sker = pl.pallas_call(
    _s_body,
    grid=(NSR, NSC),
    in_specs=[
        pl.BlockSpec((SBR, D), lambda i, j: (i, 0)),
        pl.BlockSpec((SBC, D), lambda i, j: (j, 0)),
    ],
    out_specs=pl.BlockSpec((SBR, SBC), lambda i, j: (i, j)),
    out_shape=jax.ShapeDtypeStruct((N, N), jnp.float32),
)


def kernel(x, edge_index, We1, be1, We2, be2, Wd1, bd1, Wd2, bd2,
           Ws1, bs1, Ws2, bs2):
    src = edge_index[0].astype(jnp.int32)
    dst = edge_index[1].astype(jnp.int32)
    pad = EP - src.shape[0]
    src_p = jnp.concatenate([src, jnp.zeros((pad,), jnp.int32)])
    dst_p = jnp.concatenate([dst, jnp.full((pad,), N, jnp.int32)])
    src_p = src_p.reshape(NC, NS, CH, CHUNK)
    dst_p = dst_p.reshape(NC, NS, CH, CHUNK)
    zeros_d = jnp.zeros((ZROWS, D), jnp.float32)
    zeros_16 = jnp.zeros((ZROWS, 16), jnp.float32)
    ones_16 = jnp.ones((CHUNK, 16), jnp.float32)
    be1r, be2r = be1.reshape(1, D), be2.reshape(1, D)
    bs1r, bs2r = bs1.reshape(1, D), bs2.reshape(1, D)

    _sc_degree, _sc_aggregate = _sc_kernels()
    degp = _sc_degree(dst_p, ones_16, zeros_16)

    g1 = _pre(x, We1, degp)
    p1 = _sc_aggregate(g1, src_p, dst_p, zeros_d)
    h1 = _post_relu(p1, g1, be1r, degp)
    g2 = _pre(h1, We2, degp)
    p2 = _sc_aggregate(g2, src_p, dst_p, zeros_d)
    h = _post_lin(p2, g2, be2r, degp)

    a = _acca(h, Wd1)
    t = _tker(a, bd1.reshape(1, D), Wd2, bd2.reshape(1, D))
    x_ = _xker(h, t)

    g3 = _pre(h, Ws1, degp)
    p3 = _sc_aggregate(g3, src_p, dst_p, zeros_d)
    h3 = _post_relu(p3, g3, bs1r, degp)
    g4 = _pre(h3, Ws2, degp)
    p4 = _sc_aggregate(g4, src_p, dst_p, zeros_d)
    hs = _post_lin(p4, g4, bs2r, degp)

    s_ = _sker(hs, hs)
    return (x_, s_)


# speedup vs baseline: 5.6652x; 1.2251x over previous
import functools

import jax
import jax.numpy as jnp
from jax import lax
from jax.experimental import pallas as pl
from jax.experimental.pallas import tpu as pltpu
from jax.experimental.pallas import tpu_sc as plsc

N = 10000
D = 128
NC = 2
NS = 16
CHUNK = 128
CH = 40
EP = NC * NS * CH * CHUNK
ACC_ROWS = 10240
ZROWS = ACC_ROWS // NS
DROWS = ZROWS


@functools.lru_cache(maxsize=None)
def _sc_kernels():
    mesh = plsc.VectorSubcoreMesh(core_axis_name="c", subcore_axis_name="s",
                                  num_cores=NC, num_subcores=NS)

    @functools.partial(
        pl.kernel,
        out_type=jax.ShapeDtypeStruct((NC, ACC_ROWS, 16), jnp.float32),
        mesh=mesh,
        scratch_types=[
            pltpu.VMEM((CH, CHUNK), jnp.int32),
            pltpu.VMEM((CHUNK, 16), jnp.float32),
            pltpu.VMEM_SHARED((ACC_ROWS, 16), jnp.float32),
        ],
    )
    def sc_degree(dst_hbm, ones_hbm, zeros_hbm, out_hbm, dst_v, ones_v, acc):
        c = lax.axis_index("c")
        s = lax.axis_index("s")
        pltpu.sync_copy(zeros_hbm, acc.at[pl.ds(s * ZROWS, ZROWS)])
        pltpu.sync_copy(ones_hbm, ones_v)
        pltpu.sync_copy(dst_hbm.at[c, s], dst_v)
        plsc.subcore_barrier()

        def body(j, carry):
            pltpu.sync_copy(ones_v, acc.at[dst_v.at[j]], add=True)
            return carry

        lax.fori_loop(0, CH, body, 0)
        plsc.subcore_barrier()
        pltpu.sync_copy(acc.at[pl.ds(s * DROWS, DROWS)],
                        out_hbm.at[c, pl.ds(s * DROWS, DROWS)])

    NB = 2

    @functools.partial(
        pl.kernel,
        out_type=jax.ShapeDtypeStruct((NC, ACC_ROWS, D), jnp.float32),
        mesh=mesh,
        scratch_types=[
            pltpu.VMEM((CH, CHUNK), jnp.int32),
            pltpu.VMEM((CH, CHUNK), jnp.int32),
            pltpu.VMEM((CHUNK, D), jnp.float32),
            pltpu.VMEM((CHUNK, D), jnp.float32),
            pltpu.VMEM_SHARED((ACC_ROWS, D), jnp.float32),
            pltpu.SemaphoreType.DMA,
            pltpu.SemaphoreType.DMA,
        ],
    )
    def sc_aggregate(g_hbm, src_hbm, dst_hbm, zeros_hbm, out_hbm,
                     src_v, dst_v, buf0, buf1, acc, sem0, sem1):
        c = lax.axis_index("c")
        s = lax.axis_index("s")
        pltpu.sync_copy(zeros_hbm, acc.at[pl.ds(s * ZROWS, ZROWS)])
        pltpu.sync_copy(src_hbm.at[c, s], src_v)
        pltpu.sync_copy(dst_hbm.at[c, s], dst_v)
        plsc.subcore_barrier()

        bufs = (buf0, buf1)
        UNROLL = 8

        def body(i, carry):
            j = i * UNROLL
            sdesc = None
            for k in range(UNROLL):
                pltpu.async_copy(
                    g_hbm.at[src_v.at[j + k]], bufs[k % 2], sem0).wait()
                if sdesc is not None:
                    sdesc.wait()
                sdesc = pltpu.async_copy(
                    bufs[k % 2], acc.at[dst_v.at[j + k]], sem1, add=True)
            sdesc.wait()
            return carry

        lax.fori_loop(0, CH // UNROLL, body, 0)
        plsc.subcore_barrier()
        pltpu.sync_copy(acc.at[pl.ds(s * DROWS, DROWS)],
                        out_hbm.at[c, pl.ds(s * DROWS, DROWS)])

    return sc_degree, sc_aggregate


BR = 1000
NRB = N // BR


def _dinv_block(degp_ref):
    deg = degp_ref[0, :, 0:1] + degp_ref[1, :, 0:1] + 1.0
    return lax.rsqrt(deg)


def _pre_body(h_ref, w_ref, degp_ref, g_ref):
    mm = jnp.dot(h_ref[...], w_ref[...], preferred_element_type=jnp.float32)
    g_ref[...] = _dinv_block(degp_ref) * mm


_pre = pl.pallas_call(
    _pre_body,
    grid=(NRB,),
    in_specs=[
        pl.BlockSpec((BR, D), lambda i: (i, 0)),
        pl.BlockSpec((D, D), lambda i: (0, 0)),
        pl.BlockSpec((NC, BR, 16), lambda i: (0, i, 0)),
    ],
    out_specs=pl.BlockSpec((BR, D), lambda i: (i, 0)),
    out_shape=jax.ShapeDtypeStruct((N, D), jnp.float32),
)


def _post_body(accp_ref, g_ref, b_ref, degp_ref, o_ref, *, relu):
    dinv = _dinv_block(degp_ref)
    v = dinv * (accp_ref[0] + accp_ref[1] + g_ref[...]) + b_ref[...]
    o_ref[...] = jnp.maximum(v, 0.0) if relu else v


def _make_post(relu):
    return pl.pallas_call(
        functools.partial(_post_body, relu=relu),
        grid=(NRB,),
        in_specs=[
            pl.BlockSpec((NC, BR, D), lambda i: (0, i, 0)),
            pl.BlockSpec((BR, D), lambda i: (i, 0)),
            pl.BlockSpec((1, D), lambda i: (0, 0)),
            pl.BlockSpec((NC, BR, 16), lambda i: (0, i, 0)),
        ],
        out_specs=pl.BlockSpec((BR, D), lambda i: (i, 0)),
        out_shape=jax.ShapeDtypeStruct((N, D), jnp.float32),
    )


_post_relu = _make_post(True)
_post_lin = _make_post(False)


def _acca_body(h_ref, wd1_ref, a_ref):
    @pl.when(pl.program_id(0) == 0)
    def _init():
        a_ref[...] = jnp.zeros_like(a_ref)

    a_ref[...] += lax.dot_general(
        h_ref[...], wd1_ref[...], (((0,), (0,)), ((), ())),
        preferred_element_type=jnp.float32)


_acca = pl.pallas_call(
    _acca_body,
    grid=(NRB,),
    in_specs=[
        pl.BlockSpec((BR, D), lambda i: (i, 0)),
        pl.BlockSpec((BR, D), lambda i: (i, 0)),
    ],
    out_specs=pl.BlockSpec((D, D), lambda i: (0, 0)),
    out_shape=jax.ShapeDtypeStruct((D, D), jnp.float32),
)


def _t_body(a_ref, bd1_ref, wd2_ref, bd2_ref, t_ref):
    u = jnp.maximum(a_ref[...] + bd1_ref[...], 0.0)
    t_ref[...] = jnp.dot(u, wd2_ref[...],
                         preferred_element_type=jnp.float32) + bd2_ref[...]


_tker = pl.pallas_call(
    _t_body,
    out_shape=jax.ShapeDtypeStruct((D, D), jnp.float32),
)


def _x_body(h_ref, t_ref, x_ref):
    x_ref[...] = jnp.dot(h_ref[...], t_ref[...],
                         preferred_element_type=jnp.float32)


_xker = pl.pallas_call(
    _x_body,
    grid=(NRB,),
    in_specs=[
        pl.BlockSpec((BR, D), lambda i: (i, 0)),
        pl.BlockSpec((D, D), lambda i: (0, 0)),
    ],
    out_specs=pl.BlockSpec((BR, D), lambda i: (i, 0)),
    out_shape=jax.ShapeDtypeStruct((N, D), jnp.float32),
)


SBR = 1000
SBC = 1024
NSR = N // SBR
NSC = -(-N // SBC)


def _s_body(hsr_ref, hsc_ref, o_ref):
    logits = lax.dot_general(
        hsr_ref[...], hsc_ref[...], (((1,), (1,)), ((), ())),
        preferred_element_type=jnp.float32)
    o_ref[...] = jax.nn.sigmoid(logits)


_sker = pl.pallas_call(
    _s_body,
    grid=(NSR, NSC),
    in_specs=[
        pl.BlockSpec((SBR, D), lambda i, j: (i, 0)),
        pl.BlockSpec((SBC, D), lambda i, j: (j, 0)),
    ],
    out_specs=pl.BlockSpec((SBR, SBC), lambda i, j: (i, j)),
    out_shape=jax.ShapeDtypeStruct((N, N), jnp.float32),
)


def kernel(x, edge_index, We1, be1, We2, be2, Wd1, bd1, Wd2, bd2,
           Ws1, bs1, Ws2, bs2):
    src = edge_index[0].astype(jnp.int32)
    dst = edge_index[1].astype(jnp.int32)
    pad = EP - src.shape[0]
    src_p = jnp.concatenate([src, jnp.zeros((pad,), jnp.int32)])
    dst_p = jnp.concatenate([dst, jnp.full((pad,), N, jnp.int32)])
    src_p = src_p.reshape(NC, NS, CH, CHUNK)
    dst_p = dst_p.reshape(NC, NS, CH, CHUNK)
    zeros_d = jnp.zeros((ZROWS, D), jnp.float32)
    zeros_16 = jnp.zeros((ZROWS, 16), jnp.float32)
    ones_16 = jnp.ones((CHUNK, 16), jnp.float32)
    be1r, be2r = be1.reshape(1, D), be2.reshape(1, D)
    bs1r, bs2r = bs1.reshape(1, D), bs2.reshape(1, D)

    _sc_degree, _sc_aggregate = _sc_kernels()
    degp = _sc_degree(dst_p, ones_16, zeros_16)

    g1 = _pre(x, We1, degp)
    p1 = _sc_aggregate(g1, src_p, dst_p, zeros_d)
    h1 = _post_relu(p1, g1, be1r, degp)
    g2 = _pre(h1, We2, degp)
    p2 = _sc_aggregate(g2, src_p, dst_p, zeros_d)
    h = _post_lin(p2, g2, be2r, degp)

    a = _acca(h, Wd1)
    t = _tker(a, bd1.reshape(1, D), Wd2, bd2.reshape(1, D))
    x_ = _xker(h, t)

    g3 = _pre(h, Ws1, degp)
    p3 = _sc_aggregate(g3, src_p, dst_p, zeros_d)
    h3 = _post_relu(p3, g3, bs1r, degp)
    g4 = _pre(h3, Ws2, degp)
    p4 = _sc_aggregate(g4, src_p, dst_p, zeros_d)
    hs = _post_lin(p4, g4, bs2r, degp)

    s_ = _sker(hs, hs)
    return (x_, s_)
